# Initial kernel scaffold; baseline (speedup 1.0000x reference)
import jax
import jax.numpy as jnp
from jax.experimental import pallas as pl


def kernel(output, groundtruth, k):
    raise NotImplementedError("write your pallas kernel here")


# speedup vs baseline: 18.5373x; 18.5373x over previous
import jax
import jax.numpy as jnp
from jax.experimental import pallas as pl
from jax.experimental.pallas import tpu as pltpu


def _select_body(k_ref, o_ref, g_ref, out_ref):
    d = o_ref[...] - g_ref[...]
    loss = d * d
    bits = jax.lax.bitcast_convert_type(loss, jnp.int32)
    kk = k_ref[0]

    def step(i, prefix):
        cand = prefix | jnp.left_shift(jnp.int32(1), jnp.int32(30) - i)
        cnt = jnp.sum((bits < cand).astype(jnp.int32))
        return jnp.where(cnt >= kk, prefix, cand)

    v = jax.lax.fori_loop(0, 31, step, jnp.int32(0))
    mask = bits < v
    cnt_less = jnp.sum(mask.astype(jnp.int32))
    sum_less = jnp.sum(jnp.where(mask, loss, jnp.float32(0.0)))
    vf = jax.lax.bitcast_convert_type(v, jnp.float32)
    kf = kk.astype(jnp.float32)
    out_ref[0] = (sum_less + (kk - cnt_less).astype(jnp.float32) * vf) / kf


def kernel(output, groundtruth, k):
    karr = jnp.asarray(k, jnp.int32).reshape(1)
    out = pl.pallas_call(
        _select_body,
        out_shape=jax.ShapeDtypeStruct((1,), jnp.float32),
        in_specs=[
            pl.BlockSpec(memory_space=pltpu.SMEM),
            pl.BlockSpec(memory_space=pltpu.VMEM),
            pl.BlockSpec(memory_space=pltpu.VMEM),
        ],
        out_specs=pl.BlockSpec(memory_space=pltpu.SMEM),
    )(karr, output, groundtruth)
    return out[0]
